# baseline (speedup 1.0000x reference)
import functools

import jax
import jax.numpy as jnp
from jax.experimental import pallas as pl
from jax.experimental.pallas import tpu as pltpu

PRE_NMS_THRESH = 0.05
PRE_NMS_TOP_N = 1000
NMS_THRESH = 0.6
FPN_POST_NMS_TOP_N = 100
NUM_CLASSES = 81


def _score_body(cls_ref, ctr_ref, out_ref):
    c = cls_ref[...]
    t = ctr_ref[...]
    sc = jax.nn.sigmoid(c)
    st = jax.nn.sigmoid(t)
    out_ref[...] = jnp.where(sc > PRE_NMS_THRESH, sc * st, 0.0)


def _scores_flat(box_cls, centerness):
    N, C, H, W = box_cls.shape
    HW = H * W
    R = HW * C // 128
    t_cls = jnp.transpose(box_cls, (0, 2, 3, 1)).reshape(N, R, 128)
    t_ctr = jnp.repeat(
        jnp.transpose(centerness, (0, 2, 3, 1)).reshape(N, HW), C, axis=1
    ).reshape(N, R, 128)
    out = pl.pallas_call(
        _score_body,
        out_shape=jax.ShapeDtypeStruct((N, R, 128), jnp.float32),
        grid=(N,),
        in_specs=[
            pl.BlockSpec((1, R, 128), lambda n: (n, 0, 0)),
            pl.BlockSpec((1, R, 128), lambda n: (n, 0, 0)),
        ],
        out_specs=pl.BlockSpec((1, R, 128), lambda n: (n, 0, 0)),
    )(t_cls, t_ctr)
    return out.reshape(N, HW * C)


def _pairwise_iou(boxes):
    x0, y0, x1, y1 = boxes[:, 0], boxes[:, 1], boxes[:, 2], boxes[:, 3]
    area = jnp.clip(x1 - x0, 0.0) * jnp.clip(y1 - y0, 0.0)
    xx0 = jnp.maximum(x0[:, None], x0[None, :])
    yy0 = jnp.maximum(y0[:, None], y0[None, :])
    xx1 = jnp.minimum(x1[:, None], x1[None, :])
    yy1 = jnp.minimum(y1[:, None], y1[None, :])
    inter = jnp.clip(xx1 - xx0, 0.0) * jnp.clip(yy1 - yy0, 0.0)
    union = area[:, None] + area[None, :] - inter
    return inter / jnp.maximum(union, 1e-9)


def _nms_keep(boxes, valid, thresh):
    K = boxes.shape[0]
    iou = _pairwise_iou(boxes)
    idxs = jnp.arange(K)

    def body(i, keep):
        sup = (iou[i] > thresh) & (idxs > i) & keep[i]
        return keep & (~sup)

    return jax.lax.fori_loop(0, K, body, valid)


def kernel(locations, box_cls, box_regression, centerness, image_sizes):
    N, C, H, W = box_cls.shape
    flat = _scores_flat(box_cls, centerness)
    reg = jnp.transpose(box_regression, (0, 2, 3, 1)).reshape(N, -1, 4)
    vals, idx = jax.lax.top_k(flat, PRE_NMS_TOP_N)
    loc_idx = idx // C
    labels = (idx % C) + 1
    reg_g = jnp.take_along_axis(reg, loc_idx[:, :, None], axis=1)
    loc_xy = jnp.take(locations, loc_idx, axis=0)
    x0 = loc_xy[..., 0] - reg_g[..., 0]
    y0 = loc_xy[..., 1] - reg_g[..., 1]
    x1 = loc_xy[..., 0] + reg_g[..., 2]
    y1 = loc_xy[..., 1] + reg_g[..., 3]
    h_img = image_sizes[:, 0].astype(jnp.float32)[:, None]
    w_img = image_sizes[:, 1].astype(jnp.float32)[:, None]
    x0 = jnp.clip(x0, 0.0, w_img - 1.0)
    x1 = jnp.clip(x1, 0.0, w_img - 1.0)
    y0 = jnp.clip(y0, 0.0, h_img - 1.0)
    y1 = jnp.clip(y1, 0.0, h_img - 1.0)
    det_scores = jnp.sqrt(jnp.maximum(vals, 1e-12))
    valid = (vals > 0) & (x1 - x0 + 1.0 >= 0.0) & (y1 - y0 + 1.0 >= 0.0)
    boxes = jnp.stack([x0, y0, x1, y1], axis=-1)
    offset = labels.astype(jnp.float32) * 4096.0
    boxes_off = boxes + offset[..., None]
    keep = jax.vmap(lambda b, v: _nms_keep(b, v, NMS_THRESH))(boxes_off, valid)
    kept_scores = jnp.where(keep, det_scores, -1.0)
    top_scores, top_idx = jax.lax.top_k(kept_scores, FPN_POST_NMS_TOP_N)
    out_boxes = jnp.take_along_axis(boxes, top_idx[:, :, None], axis=1)
    out_labels = jnp.take_along_axis(labels, top_idx, axis=1).astype(jnp.int32)
    return out_boxes, top_scores, out_labels


# speedup vs baseline: 1.3101x; 1.3101x over previous
import functools

import jax
import jax.numpy as jnp
from jax.experimental import pallas as pl
from jax.experimental.pallas import tpu as pltpu

PRE_NMS_THRESH = 0.05
PRE_NMS_TOP_N = 1000
NMS_THRESH = 0.6
FPN_POST_NMS_TOP_N = 100
NUM_CLASSES = 81
K = 1024


def _score_body(cls_ref, ctr_ref, out_ref):
    c = cls_ref[...]
    t = ctr_ref[...]
    sc = jax.nn.sigmoid(c)
    st = jax.nn.sigmoid(t)
    out_ref[...] = jnp.where(sc > PRE_NMS_THRESH, sc * st, 0.0)


def _scores_flat(box_cls, centerness):
    N, C, H, W = box_cls.shape
    HW = H * W
    R = HW * C // 128
    t_cls = jnp.transpose(box_cls, (0, 2, 3, 1)).reshape(N, R, 128)
    t_ctr = jnp.repeat(
        jnp.transpose(centerness, (0, 2, 3, 1)).reshape(N, HW), C, axis=1
    ).reshape(N, R, 128)
    out = pl.pallas_call(
        _score_body,
        out_shape=jax.ShapeDtypeStruct((N, R, 128), jnp.float32),
        grid=(N,),
        in_specs=[
            pl.BlockSpec((1, R, 128), lambda n: (n, 0, 0)),
            pl.BlockSpec((1, R, 128), lambda n: (n, 0, 0)),
        ],
        out_specs=pl.BlockSpec((1, R, 128), lambda n: (n, 0, 0)),
    )(t_cls, t_ctr)
    return out.reshape(N, HW * C)


def _nms_body(cols_ref, rows_ref, out_ref):
    A = cols_ref[0]
    B = rows_ref[0]
    f32 = jnp.float32

    x0c, y0c, x1c, y1c = A[:, 0:1], A[:, 1:2], A[:, 2:3], A[:, 3:4]
    labc, validc = A[:, 4:5], A[:, 5:6]
    x0r, y0r, x1r, y1r = B[0:1, :], B[1:2, :], B[2:3, :], B[3:4, :]
    labr, validr, detr = B[4:5, :], B[5:6, :], B[6:7, :]

    offc = labc * 4096.0
    offr = labr * 4096.0
    ax0c, ay0c, ax1c, ay1c = x0c + offc, y0c + offc, x1c + offc, y1c + offc
    ax0r, ay0r, ax1r, ay1r = x0r + offr, y0r + offr, x1r + offr, y1r + offr

    areac = jnp.clip(ax1c - ax0c, 0.0) * jnp.clip(ay1c - ay0c, 0.0)
    arear = jnp.clip(ax1r - ax0r, 0.0) * jnp.clip(ay1r - ay0r, 0.0)
    xx0 = jnp.maximum(ax0c, ax0r)
    yy0 = jnp.maximum(ay0c, ay0r)
    xx1 = jnp.minimum(ax1c, ax1r)
    yy1 = jnp.minimum(ay1c, ay1r)
    inter = jnp.clip(xx1 - xx0, 0.0) * jnp.clip(yy1 - yy0, 0.0)
    union = areac + arear - inter
    iou = inter / jnp.maximum(union, 1e-9)

    ri = jax.lax.broadcasted_iota(jnp.int32, (K, K), 0)
    ci = jax.lax.broadcasted_iota(jnp.int32, (K, K), 1)
    hot = iou > NMS_THRESH
    n1 = jnp.where(hot & (ci < ri), 1.0, 0.0).astype(f32)
    n2 = jnp.where(hot & (ri < ci), 1.0, 0.0).astype(f32)

    def cond(c):
        return c[2]

    def body(c):
        kr, kc, _ = c
        sup_r = jax.lax.dot_general(
            kr, n2, (((1,), (0,)), ((), ())), preferred_element_type=f32
        )
        sup_c = jax.lax.dot_general(
            n1, kc, (((1,), (0,)), ((), ())), preferred_element_type=f32
        )
        nr = jnp.where((validr > 0.5) & (sup_r < 0.5), 1.0, 0.0)
        nc = jnp.where((validc > 0.5) & (sup_c < 0.5), 1.0, 0.0)
        return nr, nc, jnp.any(nr != kr)

    kr0 = jnp.where(validr > 0.5, 1.0, 0.0)
    kc0 = jnp.where(validc > 0.5, 1.0, 0.0)
    kr, kc, _ = jax.lax.while_loop(cond, body, (kr0, kc0, jnp.bool_(True)))

    lo = jnp.where(ci <= ri, 1.0, 0.0).astype(f32)
    krank = jax.lax.dot_general(
        lo, kc, (((1,), (0,)), ((), ())), preferred_element_type=f32
    )
    nkrank = jax.lax.dot_general(
        lo, 1.0 - kc, (((1,), (0,)), ((), ())), preferred_element_type=f32
    )
    total = jnp.sum(kc)
    slot = jnp.where(kc > 0.5, krank - 1.0, total + nkrank - 1.0)
    slot_i = slot.astype(jnp.int32)
    lane = jax.lax.broadcasted_iota(jnp.int32, (K, 128), 1)
    sel = jnp.where((slot_i == lane), 1.0, 0.0).astype(f32)

    scorer = detr * kr - (1.0 - kr)
    v = jnp.concatenate([scorer, x0r, y0r, x1r, y1r, labr], axis=0)
    out6 = jax.lax.dot_general(
        v, sel, (((1,), (0,)), ((), ())), preferred_element_type=f32
    )
    out_ref[0, 0:6, :] = out6
    out_ref[0, 6:8, :] = jnp.zeros((2, 128), f32)


def _nms_select(cols, rows):
    N = cols.shape[0]
    return pl.pallas_call(
        _nms_body,
        out_shape=jax.ShapeDtypeStruct((N, 8, 128), jnp.float32),
        grid=(N,),
        in_specs=[
            pl.BlockSpec((1, K, 8), lambda n: (n, 0, 0)),
            pl.BlockSpec((1, 8, K), lambda n: (n, 0, 0)),
        ],
        out_specs=pl.BlockSpec((1, 8, 128), lambda n: (n, 0, 0)),
    )(cols, rows)


def kernel(locations, box_cls, box_regression, centerness, image_sizes):
    N, C, H, W = box_cls.shape
    flat = _scores_flat(box_cls, centerness)
    reg = jnp.transpose(box_regression, (0, 2, 3, 1)).reshape(N, -1, 4)
    vals, idx = jax.lax.top_k(flat, PRE_NMS_TOP_N)
    loc_idx = idx // C
    labels = ((idx % C) + 1).astype(jnp.float32)
    reg_g = jnp.take_along_axis(reg, loc_idx[:, :, None], axis=1)
    loc_xy = jnp.take(locations, loc_idx, axis=0)
    x0 = loc_xy[..., 0] - reg_g[..., 0]
    y0 = loc_xy[..., 1] - reg_g[..., 1]
    x1 = loc_xy[..., 0] + reg_g[..., 2]
    y1 = loc_xy[..., 1] + reg_g[..., 3]
    h_img = image_sizes[:, 0].astype(jnp.float32)[:, None]
    w_img = image_sizes[:, 1].astype(jnp.float32)[:, None]
    x0 = jnp.clip(x0, 0.0, w_img - 1.0)
    x1 = jnp.clip(x1, 0.0, w_img - 1.0)
    y0 = jnp.clip(y0, 0.0, h_img - 1.0)
    y1 = jnp.clip(y1, 0.0, h_img - 1.0)
    det_scores = jnp.sqrt(jnp.maximum(vals, 1e-12))
    valid = (vals > 0) & (x1 - x0 + 1.0 >= 0.0) & (y1 - y0 + 1.0 >= 0.0)
    valid = valid.astype(jnp.float32)

    pad = K - PRE_NMS_TOP_N
    rows = jnp.stack(
        [x0, y0, x1, y1, labels, valid, det_scores, jnp.zeros_like(x0)], axis=1
    )
    rows = jnp.pad(rows, ((0, 0), (0, 0), (0, pad)))
    cols = jnp.transpose(rows, (0, 2, 1))
    out = _nms_select(cols, rows)

    top = out[:, :, :FPN_POST_NMS_TOP_N]
    out_boxes = jnp.stack([top[:, 1], top[:, 2], top[:, 3], top[:, 4]], axis=-1)
    top_scores = top[:, 0]
    out_labels = top[:, 5].astype(jnp.int32)
    return out_boxes, top_scores, out_labels
